# baseline (speedup 1.0000x reference)
import functools

import jax
import jax.numpy as jnp
from jax.experimental import pallas as pl
from jax.experimental.pallas import tpu as pltpu

TOPK = 2
BM = 128
BI = 256


def _swiglu_kernel(meta_ref, xg_ref, w11_ref, w12_ref, w2_ref, wrow_ref,
                   out_ref):
    j = pl.program_id(0)
    u = pl.program_id(1)

    tile = meta_ref[1, u]
    start = meta_ref[2, u]
    end = meta_ref[3, u]
    first = meta_ref[4, u]

    base = tile * BM
    xg = xg_ref[pl.ds(base, BM), :]
    w11 = w11_ref[0]
    w12 = w12_ref[0]
    w2 = w2_ref[0]

    g = jnp.dot(xg, w11.T, preferred_element_type=jnp.float32)
    v = jnp.dot(xg, w12.T, preferred_element_type=jnp.float32)
    h = g * jax.nn.sigmoid(g) * v
    contrib = jnp.dot(h, w2.T, preferred_element_type=jnp.float32)

    rows = base + jax.lax.broadcasted_iota(jnp.int32, (BM, 1), 0)
    mask = (rows >= start) & (rows < end)
    contrib = jnp.where(mask, contrib * wrow_ref[pl.ds(base, BM), :], 0.0)

    init = (first == 1) & (j == 0)

    @pl.when(init)
    def _():
        out_ref[pl.ds(base, BM), :] = contrib

    @pl.when(jnp.logical_not(init))
    def _():
        out_ref[pl.ds(base, BM), :] += contrib


@functools.partial(jax.jit, static_argnames=())
def kernel(x, gate_W, W11, W12, W2):
    B, S, H = x.shape
    E, I, _ = W11.shape
    xs = x.reshape(S, H)

    logits = jnp.dot(xs, gate_W.T)
    gate_scores = jax.nn.softmax(logits, axis=-1)
    top_v, top_i = jax.lax.top_k(gate_scores, TOPK)
    top_v = top_v / (jnp.sum(top_v, axis=-1, keepdims=True) + 1e-8)

    N = S * TOPK
    eid = top_i.reshape(N)
    wts = top_v.reshape(N)
    tok = jnp.repeat(jnp.arange(S, dtype=jnp.int32), TOPK)
    order = jnp.argsort(eid)
    eid_s = eid[order]
    tok_s = tok[order]
    w_s = wts[order]

    xg = jnp.take(xs, tok_s, axis=0)

    T = N // BM
    U = T + E - 1
    counts = jnp.sum(eid_s[None, :] == jnp.arange(E, dtype=eid_s.dtype)[:, None],
                     axis=1).astype(jnp.int32)
    ends = jnp.cumsum(counts)
    starts = ends - counts
    first_tile = starts // BM
    last_tile = jnp.maximum(first_tile, (ends - 1) // BM)
    nunits = jnp.where(counts > 0, last_tile - first_tile + 1, 0)
    u_off = jnp.cumsum(nunits)
    u_start = u_off - nunits

    uu = jnp.arange(U, dtype=jnp.int32)
    e_of_u = jnp.searchsorted(u_off, uu, side='right').astype(jnp.int32)
    valid = e_of_u < E
    e_clip = jnp.minimum(e_of_u, E - 1)
    tile_of_u = jnp.where(valid, first_tile[e_clip] + (uu - u_start[e_clip]),
                          T - 1).astype(jnp.int32)
    start_of_u = jnp.where(valid, starts[e_clip], 0).astype(jnp.int32)
    end_of_u = jnp.where(valid, ends[e_clip], 0).astype(jnp.int32)
    first_of_u = jnp.concatenate([
        jnp.ones((1,), jnp.bool_),
        tile_of_u[1:] != tile_of_u[:-1]]) & valid
    meta = jnp.stack([e_clip, tile_of_u, start_of_u, end_of_u,
                      first_of_u.astype(jnp.int32)])

    wcol = w_s.reshape(N, 1)

    NI = I // BI
    grid_spec = pltpu.PrefetchScalarGridSpec(
        num_scalar_prefetch=1,
        grid=(NI, U),
        in_specs=[
            pl.BlockSpec((N, H), lambda j, u, m: (0, 0)),
            pl.BlockSpec((1, BI, H), lambda j, u, m: (m[0, u], j, 0)),
            pl.BlockSpec((1, BI, H), lambda j, u, m: (m[0, u], j, 0)),
            pl.BlockSpec((1, H, BI), lambda j, u, m: (m[0, u], 0, j)),
            pl.BlockSpec((N, 1), lambda j, u, m: (0, 0)),
        ],
        out_specs=pl.BlockSpec((N, H), lambda j, u, m: (0, 0)),
    )
    out_sorted = pl.pallas_call(
        _swiglu_kernel,
        grid_spec=grid_spec,
        out_shape=jax.ShapeDtypeStruct((N, H), jnp.float32),
    )(meta, xg, W11, W12, W2, wcol)

    out = jnp.zeros((S, H), jnp.float32).at[tok_s].add(out_sorted)
    return out.reshape(B, S, H)


# speedup vs baseline: 1.3005x; 1.3005x over previous
import functools

import jax
import jax.numpy as jnp
from jax.experimental import pallas as pl
from jax.experimental.pallas import tpu as pltpu

TOPK = 2
BM = 256
BI = 256


def _swiglu_kernel(meta_ref, xg_ref, w11_ref, w12_ref, w2_ref, wrow_ref,
                   out_ref):
    j = pl.program_id(0)
    u = pl.program_id(1)

    tile = meta_ref[1, u]
    start = meta_ref[2, u]
    end = meta_ref[3, u]
    first = meta_ref[4, u]

    base = tile * BM
    xg = xg_ref[pl.ds(base, BM), :]
    w11 = w11_ref[0]
    w12 = w12_ref[0]
    w2 = w2_ref[0]

    g = jnp.dot(xg, w11.T, preferred_element_type=jnp.float32)
    v = jnp.dot(xg, w12.T, preferred_element_type=jnp.float32)
    h = g * jax.nn.sigmoid(g) * v
    contrib = jnp.dot(h, w2.T, preferred_element_type=jnp.float32)

    rows = base + jax.lax.broadcasted_iota(jnp.int32, (BM, 1), 0)
    mask = (rows >= start) & (rows < end)
    contrib = jnp.where(mask, contrib * wrow_ref[pl.ds(base, BM), :], 0.0)

    init = (first == 1) & (j == 0)

    @pl.when(init)
    def _():
        out_ref[pl.ds(base, BM), :] = contrib

    @pl.when(jnp.logical_not(init))
    def _():
        out_ref[pl.ds(base, BM), :] += contrib


@functools.partial(jax.jit, static_argnames=())
def kernel(x, gate_W, W11, W12, W2):
    B, S, H = x.shape
    E, I, _ = W11.shape
    xs = x.reshape(S, H)

    logits = jnp.dot(xs, gate_W.T)
    gate_scores = jax.nn.softmax(logits, axis=-1)
    top_v, top_i = jax.lax.top_k(gate_scores, TOPK)
    top_v = top_v / (jnp.sum(top_v, axis=-1, keepdims=True) + 1e-8)

    N = S * TOPK
    eid = top_i.reshape(N)
    wts = top_v.reshape(N)
    tok = jnp.repeat(jnp.arange(S, dtype=jnp.int32), TOPK)
    order = jnp.argsort(eid)
    eid_s = eid[order]
    tok_s = tok[order]
    w_s = wts[order]

    xg = jnp.take(xs, tok_s, axis=0)

    T = N // BM
    U = T + E - 1
    counts = jnp.sum(eid_s[None, :] == jnp.arange(E, dtype=eid_s.dtype)[:, None],
                     axis=1).astype(jnp.int32)
    ends = jnp.cumsum(counts)
    starts = ends - counts
    first_tile = starts // BM
    last_tile = jnp.maximum(first_tile, (ends - 1) // BM)
    nunits = jnp.where(counts > 0, last_tile - first_tile + 1, 0)
    u_off = jnp.cumsum(nunits)
    u_start = u_off - nunits

    uu = jnp.arange(U, dtype=jnp.int32)
    e_of_u = jnp.searchsorted(u_off, uu, side='right').astype(jnp.int32)
    valid = e_of_u < E
    e_clip = jnp.minimum(e_of_u, E - 1)
    tile_of_u = jnp.where(valid, first_tile[e_clip] + (uu - u_start[e_clip]),
                          T - 1).astype(jnp.int32)
    start_of_u = jnp.where(valid, starts[e_clip], 0).astype(jnp.int32)
    end_of_u = jnp.where(valid, ends[e_clip], 0).astype(jnp.int32)
    first_of_u = jnp.concatenate([
        jnp.ones((1,), jnp.bool_),
        tile_of_u[1:] != tile_of_u[:-1]]) & valid
    meta = jnp.stack([e_clip, tile_of_u, start_of_u, end_of_u,
                      first_of_u.astype(jnp.int32)])

    wcol = w_s.reshape(N, 1)

    NI = I // BI
    grid_spec = pltpu.PrefetchScalarGridSpec(
        num_scalar_prefetch=1,
        grid=(NI, U),
        in_specs=[
            pl.BlockSpec((N, H), lambda j, u, m: (0, 0)),
            pl.BlockSpec((1, BI, H), lambda j, u, m: (m[0, u], j, 0)),
            pl.BlockSpec((1, BI, H), lambda j, u, m: (m[0, u], j, 0)),
            pl.BlockSpec((1, H, BI), lambda j, u, m: (m[0, u], 0, j)),
            pl.BlockSpec((N, 1), lambda j, u, m: (0, 0)),
        ],
        out_specs=pl.BlockSpec((N, H), lambda j, u, m: (0, 0)),
    )
    out_sorted = pl.pallas_call(
        _swiglu_kernel,
        grid_spec=grid_spec,
        out_shape=jax.ShapeDtypeStruct((N, H), jnp.float32),
    )(meta, xg, W11, W12, W2, wcol)

    out = jnp.zeros((S, H), jnp.float32).at[tok_s].add(out_sorted)
    return out.reshape(B, S, H)


# speedup vs baseline: 1.3161x; 1.0120x over previous
import functools

import jax
import jax.numpy as jnp
from jax.experimental import pallas as pl
from jax.experimental.pallas import tpu as pltpu

TOPK = 2
BM = 256
BI = 256


def _swiglu_kernel(meta_ref, xg_ref, w11_ref, w12_ref, w2_ref, wrow_ref,
                   out_ref):
    j = pl.program_id(0)
    u = pl.program_id(1)

    tile = meta_ref[1, u]
    start = meta_ref[2, u]
    end = meta_ref[3, u]
    first = meta_ref[4, u]

    xg = xg_ref[0]
    w11 = w11_ref[0]
    w12 = w12_ref[0]
    w2 = w2_ref[0]

    g = jnp.dot(xg, w11.T, preferred_element_type=jnp.float32)
    v = jnp.dot(xg, w12.T, preferred_element_type=jnp.float32)
    h = g * jax.nn.sigmoid(g) * v
    contrib = jnp.dot(h, w2.T, preferred_element_type=jnp.float32)

    base = tile * BM
    rows = base + jax.lax.broadcasted_iota(jnp.int32, (BM, 1), 0)
    mask = (rows >= start) & (rows < end)
    contrib = jnp.where(mask, contrib * wrow_ref[0], 0.0)

    init = (first == 1) & (j == 0)

    @pl.when(init)
    def _():
        out_ref[pl.ds(base, BM), :] = contrib

    @pl.when(jnp.logical_not(init))
    def _():
        out_ref[pl.ds(base, BM), :] += contrib


@functools.partial(jax.jit, static_argnames=())
def kernel(x, gate_W, W11, W12, W2):
    B, S, H = x.shape
    E, I, _ = W11.shape
    xs = x.reshape(S, H)

    logits = jnp.dot(xs, gate_W.T)
    gate_scores = jax.nn.softmax(logits, axis=-1)
    top_v, top_i = jax.lax.top_k(gate_scores, TOPK)
    top_v = top_v / (jnp.sum(top_v, axis=-1, keepdims=True) + 1e-8)

    N = S * TOPK
    eid = top_i.reshape(N)
    wts = top_v.reshape(N)
    tok = jnp.repeat(jnp.arange(S, dtype=jnp.int32), TOPK)
    order = jnp.argsort(eid)
    eid_s = eid[order]
    tok_s = tok[order]
    w_s = wts[order]

    xg = jnp.take(xs, tok_s, axis=0)

    T = N // BM
    U = T + E - 1
    counts = jnp.sum(eid_s[None, :] == jnp.arange(E, dtype=eid_s.dtype)[:, None],
                     axis=1).astype(jnp.int32)
    ends = jnp.cumsum(counts)
    starts = ends - counts
    first_tile = starts // BM
    last_tile = jnp.maximum(first_tile, (ends - 1) // BM)
    nunits = jnp.where(counts > 0, last_tile - first_tile + 1, 0)
    u_off = jnp.cumsum(nunits)
    u_start = u_off - nunits

    uu = jnp.arange(U, dtype=jnp.int32)
    e_of_u = jnp.searchsorted(u_off, uu, side='right').astype(jnp.int32)
    valid = e_of_u < E
    e_clip = jnp.minimum(e_of_u, E - 1)
    tile_of_u = jnp.where(valid, first_tile[e_clip] + (uu - u_start[e_clip]),
                          T - 1).astype(jnp.int32)
    start_of_u = jnp.where(valid, starts[e_clip], 0).astype(jnp.int32)
    end_of_u = jnp.where(valid, ends[e_clip], 0).astype(jnp.int32)
    first_of_u = jnp.concatenate([
        jnp.ones((1,), jnp.bool_),
        tile_of_u[1:] != tile_of_u[:-1]]) & valid
    meta = jnp.stack([e_clip, tile_of_u, start_of_u, end_of_u,
                      first_of_u.astype(jnp.int32)])

    xg3 = xg.reshape(T, BM, H)
    wcol3 = w_s.reshape(T, BM, 1)

    NI = I // BI
    grid_spec = pltpu.PrefetchScalarGridSpec(
        num_scalar_prefetch=1,
        grid=(NI, U),
        in_specs=[
            pl.BlockSpec((1, BM, H), lambda j, u, m: (m[1, u], 0, 0)),
            pl.BlockSpec((1, BI, H), lambda j, u, m: (m[0, u], j, 0)),
            pl.BlockSpec((1, BI, H), lambda j, u, m: (m[0, u], j, 0)),
            pl.BlockSpec((1, H, BI), lambda j, u, m: (m[0, u], 0, j)),
            pl.BlockSpec((1, BM, 1), lambda j, u, m: (m[1, u], 0, 0)),
        ],
        out_specs=pl.BlockSpec((N, H), lambda j, u, m: (0, 0)),
    )
    out_sorted = pl.pallas_call(
        _swiglu_kernel,
        grid_spec=grid_spec,
        out_shape=jax.ShapeDtypeStruct((N, H), jnp.float32),
    )(meta, xg3, W11, W12, W2, wcol3)

    inv = jnp.argsort(order)
    out = (jnp.take(out_sorted, inv[0::TOPK], axis=0) +
           jnp.take(out_sorted, inv[1::TOPK], axis=0))
    return out.reshape(B, S, H)
